# baseline (speedup 1.0000x reference)
import functools

import jax
import jax.numpy as jnp
from jax import lax
from jax.experimental import pallas as pl
from jax.experimental.pallas import tpu as pltpu
from jax.experimental.pallas import tpu_sc as plsc

FACT_DIM = 128
D_MODEL = 1024

_NC = 2
_NS = 16
_NW = _NC * _NS

_IDX_CHUNK = 128

_N_CHUNKS = 1
_BLK = 2048


def _gather_body(table_hbm, idx_hbm, out_hbm, idx_v, rows_v, sem, b_per_w):
    wid = lax.axis_index("s") * _NC + lax.axis_index("c")
    base = wid * b_per_w
    if len(idx_hbm.shape) == 2:
        seq = idx_hbm.shape[1]
        per_row = seq // b_per_w
        row = wid // per_row
        col0 = (wid % per_row) * b_per_w
        pltpu.sync_copy(idx_hbm.at[row, pl.ds(col0, b_per_w)], idx_v)
    else:
        pltpu.sync_copy(idx_hbm.at[pl.ds(base, b_per_w)], idx_v)
    n = b_per_w // _IDX_CHUNK
    copies = []
    for j in range(n):
        copies.append(
            pltpu.async_copy(
                table_hbm.at[idx_v.at[pl.ds(j * _IDX_CHUNK, _IDX_CHUNK)]],
                rows_v.at[pl.ds(j * _IDX_CHUNK, _IDX_CHUNK)],
                sem,
            )
        )
    for c in copies:
        c.wait()
    pltpu.sync_copy(rows_v, out_hbm.at[pl.ds(base, b_per_w)])


def _sc_gather(table, idx):
    b = 1
    for d in idx.shape:
        b *= d
    b_per_w = b // _NW
    mesh = plsc.VectorSubcoreMesh(core_axis_name="c", subcore_axis_name="s")
    return pl.kernel(
        functools.partial(_gather_body, b_per_w=b_per_w),
        out_type=jax.ShapeDtypeStruct((b, FACT_DIM), jnp.float32),
        mesh=mesh,
        scratch_types=[
            pltpu.VMEM((b_per_w,), jnp.int32),
            pltpu.VMEM((b_per_w, FACT_DIM), jnp.float32),
            pltpu.SemaphoreType.DMA,
        ],
    )(table, idx)


def _matmul_first_body(x_ref, w_ref, o_ref):
    o_ref[...] = lax.dot_general(
        x_ref[...].astype(jnp.bfloat16),
        w_ref[...].astype(jnp.bfloat16),
        (((1,), (1,)), ((), ())),
        preferred_element_type=jnp.float32,
    )


def _matmul_chain_body(x_ref, w_ref, buf_ref, o_ref):
    del buf_ref
    o_ref[...] = lax.dot_general(
        x_ref[...].astype(jnp.bfloat16),
        w_ref[...].astype(jnp.bfloat16),
        (((1,), (1,)), ((), ())),
        preferred_element_type=jnp.float32,
    )


def _tc_project_chunk(rows, w, buf, total_rows, row_offset):
    n_blk = rows.shape[0] // _BLK
    blk_off = row_offset // _BLK
    out_shape = jax.ShapeDtypeStruct((total_rows, D_MODEL), jnp.float32)
    x_spec = pl.BlockSpec((_BLK, FACT_DIM), lambda i: (i, 0))
    w_spec = pl.BlockSpec((D_MODEL, FACT_DIM), lambda i: (0, 0))
    o_spec = pl.BlockSpec((_BLK, D_MODEL), lambda i, _o=blk_off: (i + _o, 0))
    if buf is None:
        return pl.pallas_call(
            _matmul_first_body,
            grid=(n_blk,),
            in_specs=[x_spec, w_spec],
            out_specs=o_spec,
            out_shape=out_shape,
        )(rows, w)
    return pl.pallas_call(
        _matmul_chain_body,
        grid=(n_blk,),
        in_specs=[x_spec, w_spec, pl.BlockSpec(memory_space=pl.ANY)],
        out_specs=o_spec,
        out_shape=out_shape,
        input_output_aliases={2: 0},
    )(rows, w, buf)


_CHUNK_SIZES = (2048, 4096, 10240)


def kernel(input_ids, token_embedding, projection_weight):
    batch, seq = input_ids.shape
    total = batch * seq
    if len(_CHUNK_SIZES) == 1:
        rows = _sc_gather(token_embedding, input_ids)
        buf = _tc_project_chunk(rows, projection_weight, None, total, 0)
        return buf.reshape(batch, seq, D_MODEL)
    idx = input_ids.reshape(total)
    starts = [0]
    for c in _CHUNK_SIZES[:-1]:
        starts.append(starts[-1] + c)
    gathered = [
        _sc_gather(token_embedding, lax.slice(idx, (s,), (s + c,)))
        for s, c in zip(starts, _CHUNK_SIZES)
    ]
    buf = None
    for g, s in zip(gathered, starts):
        buf = _tc_project_chunk(g, projection_weight, buf, total, s)
    return buf.reshape(batch, seq, D_MODEL)


# speedup vs baseline: 1.1046x; 1.1046x over previous
import functools

import jax
import jax.numpy as jnp
from jax import lax
from jax.experimental import pallas as pl
from jax.experimental.pallas import tpu as pltpu
from jax.experimental.pallas import tpu_sc as plsc

FACT_DIM = 128
D_MODEL = 1024

_NC = 2
_NS = 16
_NW = _NC * _NS

_IDX_CHUNK = 128

_BLK = 2048


def _gather_body(table_hbm, idx_hbm, out_hbm, idx_v, rows_v, gsems, osems,
                 b_per_w):
    wid = lax.axis_index("s") * _NC + lax.axis_index("c")
    base = wid * b_per_w
    seq = idx_hbm.shape[1]
    per_row = seq // b_per_w
    row = wid // per_row
    col0 = (wid % per_row) * b_per_w
    pltpu.sync_copy(idx_hbm.at[row, pl.ds(col0, b_per_w)], idx_v)
    n = b_per_w // _IDX_CHUNK
    gathers = []
    for j in range(n):
        sl = pl.ds(j * _IDX_CHUNK, _IDX_CHUNK)
        gathers.append(
            pltpu.async_copy(table_hbm.at[idx_v.at[sl]], rows_v.at[sl],
                             gsems[j])
        )
    outs = []
    for j in range(n):
        gathers[j].wait()
        sl = pl.ds(j * _IDX_CHUNK, _IDX_CHUNK)
        outs.append(
            pltpu.async_copy(rows_v.at[sl],
                             out_hbm.at[pl.ds(base + j * _IDX_CHUNK,
                                              _IDX_CHUNK)],
                             osems[j])
        )
    for o in outs:
        o.wait()


def _sc_gather(table, idx):
    b = idx.shape[0] * idx.shape[1]
    b_per_w = b // _NW
    n = b_per_w // _IDX_CHUNK
    mesh = plsc.VectorSubcoreMesh(core_axis_name="c", subcore_axis_name="s")
    return pl.kernel(
        functools.partial(_gather_body, b_per_w=b_per_w),
        out_type=jax.ShapeDtypeStruct((b, FACT_DIM), jnp.float32),
        mesh=mesh,
        scratch_types=[
            pltpu.VMEM((b_per_w,), jnp.int32),
            pltpu.VMEM((b_per_w, FACT_DIM), jnp.float32),
            [pltpu.SemaphoreType.DMA] * n,
            [pltpu.SemaphoreType.DMA] * n,
        ],
    )(table, idx)


def _matmul_body(x_ref, w_ref, o_ref):
    o_ref[...] = lax.dot_general(
        x_ref[...].astype(jnp.bfloat16),
        w_ref[...].astype(jnp.bfloat16),
        (((1,), (1,)), ((), ())),
        preferred_element_type=jnp.float32,
    )


def _tc_project(rows, w):
    b = rows.shape[0]
    return pl.pallas_call(
        _matmul_body,
        grid=(b // _BLK,),
        in_specs=[
            pl.BlockSpec((_BLK, FACT_DIM), lambda i: (i, 0)),
            pl.BlockSpec((D_MODEL, FACT_DIM), lambda i: (0, 0)),
        ],
        out_specs=pl.BlockSpec((_BLK, D_MODEL), lambda i: (i, 0)),
        out_shape=jax.ShapeDtypeStruct((b, D_MODEL), jnp.float32),
    )(rows, w)


def kernel(input_ids, token_embedding, projection_weight):
    batch, seq = input_ids.shape
    rows = _sc_gather(token_embedding, input_ids)
    out = _tc_project(rows, projection_weight)
    return out.reshape(batch, seq, D_MODEL)
